# baseline (device time: 253690 ns/iter reference)
import jax
import jax.numpy as jnp
from jax import lax
from jax.experimental import pallas as pl
from jax.experimental.pallas import tpu as pltpu

N_DEV = 32


def kernel(x, w_mat, scale_x, scale_w):
    m_per, k = x.shape
    n_sh = w_mat.shape[1]

    x8 = x.astype(jnp.float8_e4m3fn)
    w8 = w_mat.astype(jnp.float8_e4m3fn)

    def body(x_ref, w_ref, sx_ref, sw_ref, out_ref, gather_ref,
             send_sems, recv_sems):
        my = lax.axis_index("i")
        left = lax.rem(my + N_DEV - 1, N_DEV)
        right = lax.rem(my + 1, N_DEV)

        barrier_sem = pltpu.get_barrier_semaphore()
        pl.semaphore_signal(barrier_sem, inc=1, device_id=(left,),
                            device_id_type=pl.DeviceIdType.MESH)
        pl.semaphore_signal(barrier_sem, inc=1, device_id=(right,),
                            device_id_type=pl.DeviceIdType.MESH)
        pl.semaphore_wait(barrier_sem, 2)

        scale = sx_ref[0] * sw_ref[0]

        def gemm_store(origin):
            a = gather_ref[origin]
            acc = lax.dot_general(
                a, w_ref[...], (((1,), (0,)), ((), ())),
                preferred_element_type=jnp.float32)
            y = acc * scale
            out_ref[pl.ds(origin * m_per, m_per), :] = y * jax.nn.sigmoid(y)

        gather_ref[my] = x_ref[...]
        gemm_store(my)

        def hop(h, carry):
            send_slot = lax.rem(my - h + N_DEV, N_DEV)
            recv_slot = lax.rem(my - h - 1 + 2 * N_DEV, N_DEV)
            rdma = pltpu.make_async_remote_copy(
                src_ref=gather_ref.at[send_slot],
                dst_ref=gather_ref.at[send_slot],
                send_sem=send_sems.at[h],
                recv_sem=recv_sems.at[h],
                device_id=(right,),
                device_id_type=pl.DeviceIdType.MESH,
            )
            rdma.start()
            rdma.wait()
            gemm_store(recv_slot)
            return carry

        lax.fori_loop(0, N_DEV - 1, hop, 0)

    return pl.pallas_call(
        body,
        out_shape=jax.ShapeDtypeStruct((N_DEV * m_per, n_sh), jnp.float32),
        in_specs=[
            pl.BlockSpec(memory_space=pltpu.VMEM),
            pl.BlockSpec(memory_space=pltpu.VMEM),
            pl.BlockSpec(memory_space=pltpu.SMEM),
            pl.BlockSpec(memory_space=pltpu.SMEM),
        ],
        out_specs=pl.BlockSpec(memory_space=pltpu.VMEM),
        scratch_shapes=[
            pltpu.VMEM((N_DEV, m_per, k), jnp.float8_e4m3fn),
            pltpu.SemaphoreType.DMA((N_DEV - 1,)),
            pltpu.SemaphoreType.DMA((N_DEV - 1,)),
        ],
        compiler_params=pltpu.CompilerParams(collective_id=0),
    )(x8, w8, scale_x, scale_w)


# device time: 133521 ns/iter; 1.9000x vs baseline; 1.9000x over previous
import numpy as np

import jax
import jax.numpy as jnp
from jax import lax
from jax.experimental import pallas as pl
from jax.experimental.pallas import tpu as pltpu

N_DEV = 32
CW_HOPS = N_DEV // 2
CCW_HOPS = N_DEV - 1 - CW_HOPS


def _ring_order() -> np.ndarray:
    import distributed_mesh_v7x as dm

    try:
        mesh = dm.get_mesh("i", N_DEV)
        devs = list(mesh.devices)
        coords = [tuple(d.coords) for d in devs]
        to_logical = {c: i for i, c in enumerate(coords)}
        xs = sorted({c[0] for c in coords})
        ys = sorted({c[1] for c in coords})
        zs = sorted({c[2] for c in coords})
        if len(xs) != 2 or set(coords) != {
            (x, y, z) for x in xs for y in ys for z in zs
        }:
            return np.arange(N_DEV, dtype=np.int32)
        path_yz = []
        for zi, z in enumerate(zs):
            row = ys if zi % 2 == 0 else list(reversed(ys))
            path_yz.extend((y, z) for y in row)
        ring_coords = [(xs[0], y, z) for (y, z) in path_yz]
        ring_coords += [(xs[1], y, z) for (y, z) in reversed(path_yz)]
        return np.array([to_logical[c] for c in ring_coords], dtype=np.int32)
    except Exception:
        return np.arange(N_DEV, dtype=np.int32)


def kernel(x, w_mat, scale_x, scale_w):
    m_per, k = x.shape
    n_sh = w_mat.shape[1]

    x8 = x.astype(jnp.float8_e4m3fn)
    w8 = w_mat.astype(jnp.float8_e4m3fn)

    ring = _ring_order()
    pos_of = np.empty(N_DEV, dtype=np.int32)
    pos_of[ring] = np.arange(N_DEV, dtype=np.int32)
    ring_j = jnp.asarray(ring)

    my = lax.axis_index("i")
    r = jnp.take(jnp.asarray(pos_of), my)
    cw_chain = jnp.take(ring_j, jnp.mod(r - jnp.arange(CW_HOPS + 1), N_DEV))
    ccw_chain = jnp.take(ring_j, jnp.mod(r + jnp.arange(CCW_HOPS + 1), N_DEV))
    neighbors = jnp.stack([
        jnp.take(ring_j, jnp.mod(r - 1, N_DEV)),
        jnp.take(ring_j, jnp.mod(r + 1, N_DEV)),
    ]).astype(jnp.int32)

    def body(x_ref, w_ref, sx_ref, sw_ref, cw_ref, ccw_ref, nbr_ref,
             out_ref, gather_ref,
             cw_send_sems, cw_recv_sems, ccw_send_sems, ccw_recv_sems):
        left = nbr_ref[0]
        right = nbr_ref[1]
        me = cw_ref[0]

        barrier_sem = pltpu.get_barrier_semaphore()
        pl.semaphore_signal(barrier_sem, inc=1, device_id=(left,),
                            device_id_type=pl.DeviceIdType.MESH)
        pl.semaphore_signal(barrier_sem, inc=1, device_id=(right,),
                            device_id_type=pl.DeviceIdType.MESH)
        pl.semaphore_wait(barrier_sem, 2)

        scale = sx_ref[0] * sw_ref[0]

        def gemm_store(origin):
            a = gather_ref[origin]
            acc = lax.dot_general(
                a, w_ref[...], (((1,), (0,)), ((), ())),
                preferred_element_type=jnp.float32)
            y = acc * scale
            out_ref[pl.ds(origin * m_per, m_per), :] = y * jax.nn.sigmoid(y)

        def descriptor(origin, send_sem, recv_sem, target):
            return pltpu.make_async_remote_copy(
                src_ref=gather_ref.at[origin],
                dst_ref=gather_ref.at[origin],
                send_sem=send_sem,
                recv_sem=recv_sem,
                device_id=(target,),
                device_id_type=pl.DeviceIdType.MESH,
            )

        def send(origin, send_sem, recv_sem, target):
            rdma = descriptor(origin, send_sem, recv_sem, target)
            rdma.start()
            return rdma

        gather_ref[me] = x_ref[...]
        cw_rdmas = [send(me, cw_send_sems.at[0], cw_recv_sems.at[0], right)]
        ccw_rdmas = [send(me, ccw_send_sems.at[0], ccw_recv_sems.at[0], left)]
        gemm_store(me)

        for t in range(CW_HOPS):
            cw_slot = cw_ref[t + 1]
            cw = descriptor(cw_slot, cw_send_sems.at[t],
                            cw_recv_sems.at[t], right)
            cw.wait_recv()
            if t + 1 < CW_HOPS:
                cw_rdmas.append(
                    send(cw_slot, cw_send_sems.at[t + 1],
                         cw_recv_sems.at[t + 1], right))
            if t < CCW_HOPS:
                ccw_slot = ccw_ref[t + 1]
                ccw = descriptor(ccw_slot, ccw_send_sems.at[t],
                                 ccw_recv_sems.at[t], left)
                ccw.wait_recv()
                if t + 1 < CCW_HOPS:
                    ccw_rdmas.append(
                        send(ccw_slot, ccw_send_sems.at[t + 1],
                             ccw_recv_sems.at[t + 1], left))
            gemm_store(cw_slot)
            if t < CCW_HOPS:
                gemm_store(ccw_slot)

        for rd in cw_rdmas:
            rd.wait_send()
        for rd in ccw_rdmas:
            rd.wait_send()

    return pl.pallas_call(
        body,
        out_shape=jax.ShapeDtypeStruct((N_DEV * m_per, n_sh), jnp.float32),
        in_specs=[
            pl.BlockSpec(memory_space=pltpu.VMEM),
            pl.BlockSpec(memory_space=pltpu.VMEM),
            pl.BlockSpec(memory_space=pltpu.SMEM),
            pl.BlockSpec(memory_space=pltpu.SMEM),
            pl.BlockSpec(memory_space=pltpu.SMEM),
            pl.BlockSpec(memory_space=pltpu.SMEM),
            pl.BlockSpec(memory_space=pltpu.SMEM),
        ],
        out_specs=pl.BlockSpec(memory_space=pltpu.VMEM),
        scratch_shapes=[
            pltpu.VMEM((N_DEV, m_per, k), jnp.float8_e4m3fn),
            pltpu.SemaphoreType.DMA((CW_HOPS,)),
            pltpu.SemaphoreType.DMA((CW_HOPS,)),
            pltpu.SemaphoreType.DMA((CCW_HOPS,)),
            pltpu.SemaphoreType.DMA((CCW_HOPS,)),
        ],
        compiler_params=pltpu.CompilerParams(collective_id=0),
    )(x8, w8, scale_x, scale_w, cw_chain, ccw_chain, neighbors)


# device time: 107330 ns/iter; 2.3636x vs baseline; 1.2440x over previous
import numpy as np

import jax
import jax.numpy as jnp
from jax import lax
from jax.experimental import pallas as pl
from jax.experimental.pallas import tpu as pltpu

N_DEV = 32
CW_HOPS = N_DEV // 2
CCW_HOPS = N_DEV - 1 - CW_HOPS
N_SUB = 2


def _ring_order() -> np.ndarray:
    import distributed_mesh_v7x as dm

    try:
        mesh = dm.get_mesh("i", N_DEV)
        devs = list(mesh.devices)
        coords = [tuple(d.coords) for d in devs]
        to_logical = {c: i for i, c in enumerate(coords)}
        xs = sorted({c[0] for c in coords})
        ys = sorted({c[1] for c in coords})
        zs = sorted({c[2] for c in coords})
        if len(xs) != 2 or set(coords) != {
            (x, y, z) for x in xs for y in ys for z in zs
        }:
            return np.arange(N_DEV, dtype=np.int32)
        path_yz = []
        for zi, z in enumerate(zs):
            row = ys if zi % 2 == 0 else list(reversed(ys))
            path_yz.extend((y, z) for y in row)
        ring_coords = [(xs[0], y, z) for (y, z) in path_yz]
        ring_coords += [(xs[1], y, z) for (y, z) in reversed(path_yz)]
        return np.array([to_logical[c] for c in ring_coords], dtype=np.int32)
    except Exception:
        return np.arange(N_DEV, dtype=np.int32)


def kernel(x, w_mat, scale_x, scale_w):
    m_per, k = x.shape
    n_sh = w_mat.shape[1]

    x8 = x.astype(jnp.float8_e4m3fn)
    w8 = w_mat.astype(jnp.float8_e4m3fn)

    ring = _ring_order()
    pos_of = np.empty(N_DEV, dtype=np.int32)
    pos_of[ring] = np.arange(N_DEV, dtype=np.int32)
    ring_j = jnp.asarray(ring)

    my = lax.axis_index("i")
    r = jnp.take(jnp.asarray(pos_of), my)
    cw_chain = jnp.take(ring_j, jnp.mod(r - jnp.arange(CW_HOPS + 1), N_DEV))
    ccw_chain = jnp.take(ring_j, jnp.mod(r + jnp.arange(CCW_HOPS + 1), N_DEV))
    neighbors = jnp.stack([
        jnp.take(ring_j, jnp.mod(r - 1, N_DEV)),
        jnp.take(ring_j, jnp.mod(r + 1, N_DEV)),
    ]).astype(jnp.int32)

    def body(x_ref, w_ref, sx_ref, sw_ref, cw_ref, ccw_ref, nbr_ref,
             out_ref, gather_ref,
             cw_send_sems, cw_recv_sems, ccw_send_sems, ccw_recv_sems):
        left = nbr_ref[0]
        right = nbr_ref[1]
        me = cw_ref[0]

        barrier_sem = pltpu.get_barrier_semaphore()
        pl.semaphore_signal(barrier_sem, inc=1, device_id=(left,),
                            device_id_type=pl.DeviceIdType.MESH)
        pl.semaphore_signal(barrier_sem, inc=1, device_id=(right,),
                            device_id_type=pl.DeviceIdType.MESH)
        pl.semaphore_wait(barrier_sem, 2)

        scale = sx_ref[0] * sw_ref[0]

        m_sub = m_per // N_SUB

        def gemm_store(origin):
            a = gather_ref[origin]
            acc = lax.dot_general(
                a, w_ref[...], (((1,), (0,)), ((), ())),
                preferred_element_type=jnp.float32)
            y = acc * scale
            out_ref[pl.ds(origin * m_per, m_per), :] = y * jax.nn.sigmoid(y)

        def descriptor(origin, sub, send_sem, recv_sem, target):
            return pltpu.make_async_remote_copy(
                src_ref=gather_ref.at[origin, pl.ds(sub * m_sub, m_sub)],
                dst_ref=gather_ref.at[origin, pl.ds(sub * m_sub, m_sub)],
                send_sem=send_sem,
                recv_sem=recv_sem,
                device_id=(target,),
                device_id_type=pl.DeviceIdType.MESH,
            )

        def send(origin, sub, send_sem, recv_sem, target):
            rdma = descriptor(origin, sub, send_sem, recv_sem, target)
            rdma.start()
            return rdma

        gather_ref[me] = x_ref[...]
        cw_rdmas = [
            send(me, s, cw_send_sems.at[0, s], cw_recv_sems.at[0, s], right)
            for s in range(N_SUB)
        ]
        ccw_rdmas = [
            send(me, s, ccw_send_sems.at[0, s], ccw_recv_sems.at[0, s], left)
            for s in range(N_SUB)
        ]
        gemm_store(me)

        for t in range(CW_HOPS):
            cw_slot = cw_ref[t + 1]
            ccw_slot = ccw_ref[t + 1] if t < CCW_HOPS else None
            for s in range(N_SUB):
                cw = descriptor(cw_slot, s, cw_send_sems.at[t, s],
                                cw_recv_sems.at[t, s], right)
                cw.wait_recv()
                if t + 1 < CW_HOPS:
                    cw_rdmas.append(
                        send(cw_slot, s, cw_send_sems.at[t + 1, s],
                             cw_recv_sems.at[t + 1, s], right))
                if t < CCW_HOPS:
                    ccw = descriptor(ccw_slot, s, ccw_send_sems.at[t, s],
                                     ccw_recv_sems.at[t, s], left)
                    ccw.wait_recv()
                    if t + 1 < CCW_HOPS:
                        ccw_rdmas.append(
                            send(ccw_slot, s, ccw_send_sems.at[t + 1, s],
                                 ccw_recv_sems.at[t + 1, s], left))
            gemm_store(cw_slot)
            if t < CCW_HOPS:
                gemm_store(ccw_slot)

        for rd in cw_rdmas:
            rd.wait_send()
        for rd in ccw_rdmas:
            rd.wait_send()

    return pl.pallas_call(
        body,
        out_shape=jax.ShapeDtypeStruct((N_DEV * m_per, n_sh), jnp.float32),
        in_specs=[
            pl.BlockSpec(memory_space=pltpu.VMEM),
            pl.BlockSpec(memory_space=pltpu.VMEM),
            pl.BlockSpec(memory_space=pltpu.SMEM),
            pl.BlockSpec(memory_space=pltpu.SMEM),
            pl.BlockSpec(memory_space=pltpu.SMEM),
            pl.BlockSpec(memory_space=pltpu.SMEM),
            pl.BlockSpec(memory_space=pltpu.SMEM),
        ],
        out_specs=pl.BlockSpec(memory_space=pltpu.VMEM),
        scratch_shapes=[
            pltpu.VMEM((N_DEV, m_per, k), jnp.float8_e4m3fn),
            pltpu.SemaphoreType.DMA((CW_HOPS, N_SUB)),
            pltpu.SemaphoreType.DMA((CW_HOPS, N_SUB)),
            pltpu.SemaphoreType.DMA((CCW_HOPS, N_SUB)),
            pltpu.SemaphoreType.DMA((CCW_HOPS, N_SUB)),
        ],
        compiler_params=pltpu.CompilerParams(collective_id=0),
    )(x8, w8, scale_x, scale_w, cw_chain, ccw_chain, neighbors)
